# baseline (speedup 1.0000x reference)
import functools
import math

import jax
import jax.numpy as jnp
from jax import lax
from jax.experimental import pallas as pl
from jax.experimental.pallas import tpu as pltpu
from jax.experimental.pallas import tpu_sc as plsc

NW = 32
LANES = 16
CH = 16
NG = 3
NO = 2
BLK = 8


@functools.cache
def _make_kernel(B, S, D):
    scale = math.sqrt(D)
    tok_w = S // NW
    n_q = tok_w // CH
    n_ch = B * n_q
    mesh = plsc.VectorSubcoreMesh(core_axis_name="c", subcore_axis_name="s")

    @functools.partial(
        pl.kernel,
        mesh=mesh,
        out_type=jax.ShapeDtypeStruct((B * S, D), jnp.float32),
        scratch_types=[
            pltpu.VMEM((B, tok_w), jnp.int32),
            pltpu.VMEM((NG, CH, D), jnp.float32),
            pltpu.VMEM((NO, CH, D), jnp.float32),
            pltpu.VMEM((2, CH, D), jnp.float32),
            pltpu.SemaphoreType.DMA,
            pltpu.SemaphoreType.DMA,
            pltpu.SemaphoreType.DMA,
            pltpu.SemaphoreType.DMA,
            pltpu.SemaphoreType.DMA,
            pltpu.SemaphoreType.DMA,
            pltpu.SemaphoreType.DMA,
        ],
    )
    def emb_kernel(x_hbm, table_hbm, pos_hbm, out_hbm,
                   idx_v, rows_v, stage_v, pos_v,
                   gs0, gs1, gs2, os0, os1, ps0, ps1):
        wid = lax.axis_index("s") * 2 + lax.axis_index("c")
        sbase = wid * tok_w
        gsems = (gs0, gs1, gs2)
        osems = (os0, os1)
        psems = (ps0, ps1)

        def load_pos(q):
            return pltpu.async_copy(
                pos_hbm.at[pl.ds(sbase + q * CH, CH)],
                pos_v.at[q % 2], psems[q % 2])

        def start_gather(ch):
            q, b = divmod(ch, B)
            return pltpu.async_copy(
                table_hbm.at[idx_v.at[b, pl.ds(q * CH, CH)]],
                rows_v.at[ch % NG], gsems[ch % NG])

        p = {0: load_pos(0), 1: load_pos(1)}
        idx_cps = [pltpu.async_copy(x_hbm.at[b, pl.ds(sbase, tok_w)],
                                    idx_v.at[b], os0)
                   for b in range(B)]
        for cp in idx_cps:
            cp.wait()
        g = {ch: start_gather(ch) for ch in range(NG)}
        o = {}
        for ch in range(n_ch):
            q, b = divmod(ch, B)
            s = ch % NG
            t = ch % NO
            if b == 0:
                p[q].wait()
            g[ch].wait()
            if ch >= NO:
                o[ch - NO].wait()

            def row_body(r, _, s=s, t=t, q=q):
                for j0 in range(0, D // LANES, BLK):
                    a = [rows_v[s, r, pl.ds((j0 + k) * LANES, LANES)]
                         for k in range(BLK)]
                    c = [pos_v[q % 2, r, pl.ds((j0 + k) * LANES, LANES)]
                         for k in range(BLK)]
                    for k in range(BLK):
                        stage_v[t, r, pl.ds((j0 + k) * LANES, LANES)] = (
                            a[k] * scale + c[k])
                return 0

            lax.fori_loop(0, CH, row_body, 0)
            if ch + NG < n_ch:
                g[ch + NG] = start_gather(ch + NG)
            if b == B - 1 and q + 2 < n_q:
                p[q + 2] = load_pos(q + 2)
            o[ch] = pltpu.async_copy(
                stage_v.at[t],
                out_hbm.at[pl.ds(b * S + sbase + q * CH, CH)], osems[t])
        for ch in range(n_ch - NO, n_ch):
            o[ch].wait()

    return emb_kernel


def kernel(x, emb_table, pos_enc):
    B, S = x.shape
    D = emb_table.shape[1]
    out = _make_kernel(B, S, D)(x, emb_table, pos_enc)
    return out.reshape(B, S, D)


# speedup vs baseline: 1.0033x; 1.0033x over previous
import functools
import math

import jax
import jax.numpy as jnp
from jax import lax
from jax.experimental import pallas as pl
from jax.experimental.pallas import tpu as pltpu
from jax.experimental.pallas import tpu_sc as plsc

NW = 32
LANES = 16
CH = 16
BLK = 8
GRP = 8


@functools.cache
def _make_kernel(B, S, D):
    scale = math.sqrt(D)
    tok_w = S // NW
    n_q = tok_w // CH
    n_ch = B * n_q
    n_grp = n_ch // GRP
    mesh = plsc.VectorSubcoreMesh(core_axis_name="c", subcore_axis_name="s")

    @functools.partial(
        pl.kernel,
        mesh=mesh,
        out_type=jax.ShapeDtypeStruct((B * S, D), jnp.float32),
        scratch_types=[
            pltpu.VMEM((B, tok_w), jnp.int32),
            pltpu.VMEM((2, CH, D), jnp.float32),
            pltpu.VMEM((2, CH, D), jnp.float32),
            pltpu.VMEM((2, CH, D), jnp.float32),
            pltpu.SemaphoreType.DMA,
            pltpu.SemaphoreType.DMA,
            pltpu.SemaphoreType.DMA,
            pltpu.SemaphoreType.DMA,
            pltpu.SemaphoreType.DMA,
            pltpu.SemaphoreType.DMA,
        ],
    )
    def emb_kernel(x_hbm, table_hbm, pos_hbm, out_hbm,
                   idx_v, rows_v, stage_v, pos_v,
                   gs0, gs1, os0, os1, ps0, ps1):
        wid = lax.axis_index("s") * 2 + lax.axis_index("c")
        sbase = wid * tok_w
        gsems = (gs0, gs1)
        osems = (os0, os1)
        psems = (ps0, ps1)

        def load_pos(q, slot):
            return pltpu.make_async_copy(
                pos_hbm.at[pl.ds(sbase + q * CH, CH)],
                pos_v.at[slot], psems[slot])

        def gather_copy(q, b, slot):
            return pltpu.make_async_copy(
                table_hbm.at[idx_v.at[b, pl.ds(q * CH, CH)]],
                rows_v.at[slot], gsems[slot])

        def out_copy(q, b, slot):
            return pltpu.make_async_copy(
                stage_v.at[slot],
                out_hbm.at[pl.ds(b * S + sbase + q * CH, CH)], osems[slot])

        load_pos(0, 0).start()
        load_pos(1, 1).start()
        idx_cps = [pltpu.async_copy(x_hbm.at[b, pl.ds(sbase, tok_w)],
                                    idx_v.at[b], os0)
                   for b in range(B)]
        for cp in idx_cps:
            cp.wait()
        gather_copy(0, 0, 0).start()
        gather_copy(0, 1, 1).start()

        def group_body(g, _):
            q0 = 2 * g
            for i in range(GRP):
                s = i % 2
                q = q0 + i // 4
                b = i % 4
                if i % 4 == 0:
                    load_pos(q, i // 4).wait()
                gather_copy(q, b, s).wait()
                if i >= 2:
                    out_copy(q0 + (i - 2) // 4, (i - 2) % 4, s).wait()
                else:
                    @pl.when(g > 0)
                    def _(q=q, i=i, s=s):
                        out_copy(q0 - 2 + (i + GRP - 2) // 4,
                                 (i + GRP - 2) % 4, s).wait()

                def row_body(r, _, s=s, t=s, ps=i // 4):
                    for j0 in range(0, D // LANES, BLK):
                        a = [rows_v[s, r, pl.ds((j0 + k) * LANES, LANES)]
                             for k in range(BLK)]
                        c = [pos_v[ps, r, pl.ds((j0 + k) * LANES, LANES)]
                             for k in range(BLK)]
                        for k in range(BLK):
                            stage_v[t, r, pl.ds((j0 + k) * LANES, LANES)] = (
                                a[k] * scale + c[k])
                    return 0

                lax.fori_loop(0, CH, row_body, 0)

                if i + 2 < GRP:
                    gather_copy(q0 + (i + 2) // 4, (i + 2) % 4, s).start()
                else:
                    @pl.when(g + 1 < n_grp)
                    def _(q0=q0, i=i, s=s):
                        gather_copy(q0 + 2 + (i + 2 - GRP) // 4,
                                    (i + 2 - GRP) % 4, s).start()
                if i == 3 or i == GRP - 1:
                    @pl.when(g + 1 < n_grp)
                    def _(q0=q0, i=i):
                        load_pos(q0 + 2 + (0 if i == 3 else 1),
                                 0 if i == 3 else 1).start()
                out_copy(q, b, s).start()
            return 0

        lax.fori_loop(0, n_grp, group_body, 0)
        out_copy(0, 0, 0).wait()
        out_copy(0, 0, 1).wait()

    return emb_kernel


def kernel(x, emb_table, pos_enc):
    B, S = x.shape
    D = emb_table.shape[1]
    out = _make_kernel(B, S, D)(x, emb_table, pos_enc)
    return out.reshape(B, S, D)


# speedup vs baseline: 1.0364x; 1.0330x over previous
import functools
import math

import jax
import jax.numpy as jnp
from jax import lax
from jax.experimental import pallas as pl
from jax.experimental.pallas import tpu as pltpu
from jax.experimental.pallas import tpu_sc as plsc

NW = 32
LANES = 16
CH = 16
BLK = 8
GRP = 8


@functools.cache
def _make_kernel(B, S, D):
    scale = math.sqrt(D)
    tok_w = S // NW
    n_q = tok_w // CH
    n_ch = B * n_q
    n_grp = n_ch // GRP
    mesh = plsc.VectorSubcoreMesh(core_axis_name="c", subcore_axis_name="s")

    @functools.partial(
        pl.kernel,
        mesh=mesh,
        out_type=jax.ShapeDtypeStruct((B * S, D), jnp.float32),
        scratch_types=[
            pltpu.VMEM((B, tok_w), jnp.int32),
            pltpu.VMEM((3, CH, D), jnp.float32),
            pltpu.VMEM((2, CH, D), jnp.float32),
            pltpu.VMEM((2, CH, D), jnp.float32),
            pltpu.SemaphoreType.DMA,
            pltpu.SemaphoreType.DMA,
            pltpu.SemaphoreType.DMA,
            pltpu.SemaphoreType.DMA,
            pltpu.SemaphoreType.DMA,
            pltpu.SemaphoreType.DMA,
            pltpu.SemaphoreType.DMA,
        ],
    )
    def emb_kernel(x_hbm, table_hbm, pos_hbm, out_hbm,
                   idx_v, rows_v, stage_v, pos_v,
                   gs0, gs1, gs2, os0, os1, ps0, ps1):
        wid = lax.axis_index("s") * 2 + lax.axis_index("c")
        sbase = wid * tok_w
        gsems = (gs0, gs1, gs2)
        osems = (os0, os1)
        psems = (ps0, ps1)

        def load_pos(q, slot):
            return pltpu.make_async_copy(
                pos_hbm.at[pl.ds(sbase + q * CH, CH)],
                pos_v.at[slot], psems[slot])

        def gather_copy(q, b, slot):
            return pltpu.make_async_copy(
                table_hbm.at[idx_v.at[b, pl.ds(q * CH, CH)]],
                rows_v.at[slot], gsems[slot])

        def out_copy(q, b, slot):
            return pltpu.make_async_copy(
                stage_v.at[slot],
                out_hbm.at[pl.ds(b * S + sbase + q * CH, CH)], osems[slot])

        load_pos(0, 0).start()
        load_pos(1, 1).start()
        idx_cps = [pltpu.async_copy(x_hbm.at[b, pl.ds(sbase, tok_w)],
                                    idx_v.at[b], os0)
                   for b in range(B)]
        for cp in idx_cps:
            cp.wait()
        gather_copy(0, 0, 0).start()
        gather_copy(0, 1, 1).start()

        def group_body(g, _):
            q0 = 2 * g
            for i in range(GRP):
                gsl = i % 3
                t = i % 2
                q = q0 + i // 4
                b = i % 4
                if i % 4 == 0:
                    load_pos(q, i // 4).wait()
                gather_copy(q, b, gsl).wait()
                if i >= 2:
                    out_copy(q0 + (i - 2) // 4, (i - 2) % 4, t).wait()
                else:
                    @pl.when(g > 0)
                    def _(q=q, i=i, t=t):
                        out_copy(q0 - 2 + (i + GRP - 2) // 4,
                                 (i + GRP - 2) % 4, t).wait()
                if i + 2 < GRP:
                    gather_copy(q0 + (i + 2) // 4, (i + 2) % 4,
                                (i + 2) % 3).start()

                def row_body(r, _, s=gsl, t=t, ps=i // 4):
                    for j0 in range(0, D // LANES, BLK):
                        a = [rows_v[s, r, pl.ds((j0 + k) * LANES, LANES)]
                             for k in range(BLK)]
                        c = [pos_v[ps, r, pl.ds((j0 + k) * LANES, LANES)]
                             for k in range(BLK)]
                        for k in range(BLK):
                            stage_v[t, r, pl.ds((j0 + k) * LANES, LANES)] = (
                                a[k] * scale + c[k])
                    return 0

                lax.fori_loop(0, CH, row_body, 0)

                if i + 2 >= GRP:
                    @pl.when(g + 1 < n_grp)
                    def _(q0=q0, i=i):
                        gather_copy(q0 + 2 + (i + 2 - GRP) // 4,
                                    (i + 2 - GRP) % 4, (i + 2 - GRP) % 3).start()
                if i == 3 or i == GRP - 1:
                    @pl.when(g + 1 < n_grp)
                    def _(q0=q0, i=i):
                        load_pos(q0 + 2 + (0 if i == 3 else 1),
                                 0 if i == 3 else 1).start()
                out_copy(q, b, t).start()
            return 0

        lax.fori_loop(0, n_grp, group_body, 0)
        out_copy(0, 0, 0).wait()
        out_copy(0, 0, 1).wait()

    return emb_kernel


def kernel(x, emb_table, pos_enc):
    B, S = x.shape
    D = emb_table.shape[1]
    out = _make_kernel(B, S, D)(x, emb_table, pos_enc)
    return out.reshape(B, S, D)
